# baseline (speedup 1.0000x reference)
import functools

import jax
import jax.numpy as jnp
from jax import lax
from jax.experimental import pallas as pl
from jax.experimental.pallas import tpu as pltpu
from jax.experimental.pallas import tpu_sc as plsc

_NC = 2
_NS = 16
_LANES = 16

_CH = 80
_GRP = 1


@functools.lru_cache(maxsize=None)
def _make_sc_aggregate(N, E, D):
    DH = D
    EPT = E // _NS
    NCHUNK = EPT // _CH
    NGROUP = NCHUNK // _GRP
    assert EPT * _NS == E and NCHUNK * _CH == EPT and NGROUP * _GRP == NCHUNK
    NP = -(-N // (_NS * 128)) * (_NS * 128)
    ROWS_PT = NP // _NS
    assert ROWS_PT % _CH == 0 and DH % _LANES == 0

    mesh = plsc.VectorSubcoreMesh(core_axis_name="c", subcore_axis_name="s",
                                  num_cores=1)

    @functools.partial(
        pl.kernel,
        out_type=jax.ShapeDtypeStruct((NP, DH), jnp.float32),
        mesh=mesh,
        scratch_types=[
            [pltpu.VMEM((_GRP, _CH), jnp.int32) for _ in range(2)],
            [pltpu.VMEM((_GRP, _CH), jnp.int32) for _ in range(2)],
            [pltpu.VMEM((_GRP, _CH), jnp.float32) for _ in range(2)],
            [pltpu.VMEM((_CH, DH), jnp.float32) for _ in range(2 * _GRP)],
            pltpu.VMEM_SHARED((NP, DH), jnp.float32),
            [pltpu.SemaphoreType.DMA for _ in range(2)],
            [pltpu.SemaphoreType.DMA for _ in range(2)],
        ],
    )
    def agg(x2_hbm, src_hbm, dst_hbm, ew_hbm, out_hbm,
            src_v, dst_v, w_v, rows, accum, gsem, ssem):
        c = lax.axis_index("c")
        s = lax.axis_index("s")

        zero = jnp.zeros((_LANES,), jnp.float32)

        def zbody(i, _):
            r = i // (DH // _LANES)
            f = i % (DH // _LANES)
            rows[0][r, pl.ds(f * _LANES, _LANES)] = zero
            return 0

        lax.fori_loop(0, _CH * (DH // _LANES), zbody, 0)
        for j in range(ROWS_PT // _CH):
            pltpu.sync_copy(rows[0],
                            accum.at[pl.ds(s * ROWS_PT + j * _CH, _CH)])
        plsc.subcore_barrier()

        del c

        def scale_chunk(rows_ref, t, wrow):
            def qbody(q, _):
                w16 = w_v[t][wrow, pl.ds(q * _LANES, _LANES)]
                for e in range(_LANES):
                    wv = lax.gather(
                        w16, jnp.full((_LANES, 1), e, jnp.int32),
                        lax.GatherDimensionNumbers(
                            offset_dims=(), collapsed_slice_dims=(0,),
                            start_index_map=(0,)),
                        slice_sizes=(1,),
                        mode=lax.GatherScatterMode.PROMISE_IN_BOUNDS)
                    i = q * _LANES + e
                    for f in range(DH // _LANES):
                        sl = pl.ds(f * _LANES, _LANES)
                        rows_ref[i, sl] = rows_ref[i, sl] * wv
                return 0

            lax.fori_loop(0, _CH // _LANES, qbody, 0)

        def stage(t, g):
            pltpu.sync_copy(src_hbm.at[s, g], src_v[t])
            pltpu.sync_copy(dst_hbm.at[s, g], dst_v[t])
            pltpu.sync_copy(ew_hbm.at[s, g], w_v[t])

        def fire_gathers(t):
            for b in range(_GRP):
                pltpu.async_copy(x2_hbm.at[src_v[t].at[b]],
                                 rows[t * _GRP + b], gsem[t])

        def drain_gathers(t):
            for b in range(_GRP):
                pltpu.make_async_copy(x2_hbm.at[src_v[t].at[b]],
                                      rows[t * _GRP + b], gsem[t]).wait()

        def scale_all(t):
            for b in range(_GRP):
                scale_chunk(rows[t * _GRP + b], t, b)

        def fire_scatters(t):
            for b in range(_GRP):
                pltpu.async_copy(rows[t * _GRP + b],
                                 accum.at[dst_v[t].at[b]], ssem[t],
                                 add=True)

        def drain_scatters(t):
            for b in range(_GRP):
                pltpu.make_async_copy(rows[t * _GRP + b],
                                      accum.at[dst_v[t].at[b]],
                                      ssem[t]).wait()

        NPAIR = (NGROUP - 1) // 2
        stage(0, 0)
        fire_gathers(0)

        def pbody(k, _):
            stage(1, 2 * k + 1)
            fire_gathers(1)
            drain_gathers(0)
            scale_all(0)
            fire_scatters(0)
            drain_gathers(1)
            scale_all(1)
            fire_scatters(1)
            drain_scatters(0)
            stage(0, 2 * k + 2)
            fire_gathers(0)
            drain_scatters(1)
            return 0

        lax.fori_loop(0, NPAIR, pbody, 0)
        drain_gathers(0)
        scale_all(0)
        fire_scatters(0)
        drain_scatters(0)

        plsc.subcore_barrier()
        pltpu.sync_copy(accum.at[pl.ds(s * ROWS_PT, ROWS_PT)],
                        out_hbm.at[pl.ds(s * ROWS_PT, ROWS_PT)])

    return agg


@functools.lru_cache(maxsize=None)
def _make_tc_post(N, D):
    def body(p0_ref, p1_ref, w_ref, o_ref):
        y = (p0_ref[...] + p1_ref[...])[:N]
        z = jnp.dot(y, w_ref[...], preferred_element_type=jnp.float32)
        mean = jnp.mean(z, axis=0, keepdims=True)
        zc = z - mean
        var = jnp.mean(zc * zc, axis=0, keepdims=True)
        zr = jnp.maximum(zc * lax.rsqrt(var + 0.001), 0.0)
        ss = jnp.sum(zr * zr)
        o_ref[...] = zr * lax.rsqrt(jnp.maximum(ss, 1e-12))

    return pl.pallas_call(
        body,
        out_shape=jax.ShapeDtypeStruct((N, D), jnp.float32),
    )


def kernel(x, edge_index, edge_weight, W):
    N, D = x.shape
    E = edge_weight.shape[0]
    E2 = E // 2
    EPT = E2 // _NS
    NCHUNK = EPT // _CH
    NGROUP = NCHUNK // _GRP
    agg = _make_sc_aggregate(N, E2, D)
    parts = []
    for h in range(2):
        sl = slice(h * E2, (h + 1) * E2)
        srch = edge_index[0, sl].reshape(_NS, NGROUP, _GRP, _CH)
        dsth = edge_index[1, sl].reshape(_NS, NGROUP, _GRP, _CH)
        ewh = edge_weight[sl].reshape(_NS, NGROUP, _GRP, _CH)
        parts.append(agg(x, srch, dsth, ewh))
    return _make_tc_post(N, D)(parts[0], parts[1], W)


# speedup vs baseline: 1.1605x; 1.1605x over previous
import functools

import jax
import jax.numpy as jnp
from jax import lax
from jax.experimental import pallas as pl
from jax.experimental.pallas import tpu as pltpu
from jax.experimental.pallas import tpu_sc as plsc

_NC = 2
_NS = 16
_LANES = 16

_CH = 80
_NSLOT = 4


@functools.lru_cache(maxsize=None)
def _make_sc_aggregate(N, E, D):
    DH = D
    EPT = E // _NS
    NCHUNK = EPT // _CH
    assert EPT * _NS == E and NCHUNK * _CH == EPT
    assert NCHUNK % _NSLOT == 2
    NP = -(-N // (_NS * 128)) * (_NS * 128)
    ROWS_PT = NP // _NS
    assert ROWS_PT % _CH == 0 and DH % _LANES == 0

    mesh = plsc.VectorSubcoreMesh(core_axis_name="c", subcore_axis_name="s",
                                  num_cores=1)

    @functools.partial(
        pl.kernel,
        out_type=jax.ShapeDtypeStruct((NP, DH), jnp.float32),
        mesh=mesh,
        scratch_types=[
            [pltpu.VMEM((1, _CH), jnp.int32) for _ in range(_NSLOT)],
            [pltpu.VMEM((1, _CH), jnp.int32) for _ in range(_NSLOT)],
            [pltpu.VMEM((1, _CH), jnp.float32) for _ in range(_NSLOT)],
            [pltpu.VMEM((_CH, DH), jnp.float32) for _ in range(_NSLOT)],
            pltpu.VMEM_SHARED((NP, DH), jnp.float32),
            [pltpu.SemaphoreType.DMA for _ in range(_NSLOT)],
            [pltpu.SemaphoreType.DMA for _ in range(_NSLOT)],
        ],
    )
    def agg(x2_hbm, src_hbm, dst_hbm, ew_hbm, out_hbm,
            src_v, dst_v, w_v, rows, accum, gsem, ssem):
        c = lax.axis_index("c")
        s = lax.axis_index("s")

        zero = jnp.zeros((_LANES,), jnp.float32)

        def zbody(i, _):
            r = i // (DH // _LANES)
            f = i % (DH // _LANES)
            rows[0][r, pl.ds(f * _LANES, _LANES)] = zero
            return 0

        lax.fori_loop(0, _CH * (DH // _LANES), zbody, 0)
        for j in range(ROWS_PT // _CH):
            pltpu.sync_copy(rows[0],
                            accum.at[pl.ds(s * ROWS_PT + j * _CH, _CH)])
        plsc.subcore_barrier()

        del c

        def scale_chunk(rows_ref, t):
            def qbody(q, _):
                w16 = w_v[t][0, pl.ds(q * _LANES, _LANES)]
                for e in range(_LANES):
                    wv = lax.gather(
                        w16, jnp.full((_LANES, 1), e, jnp.int32),
                        lax.GatherDimensionNumbers(
                            offset_dims=(), collapsed_slice_dims=(0,),
                            start_index_map=(0,)),
                        slice_sizes=(1,),
                        mode=lax.GatherScatterMode.PROMISE_IN_BOUNDS)
                    i = q * _LANES + e
                    for f in range(DH // _LANES):
                        sl = pl.ds(f * _LANES, _LANES)
                        rows_ref[i, sl] = rows_ref[i, sl] * wv
                return 0

            lax.fori_loop(0, _CH // _LANES, qbody, 0)

        def stage(t, g):
            pltpu.sync_copy(src_hbm.at[s, g], src_v[t])
            pltpu.sync_copy(dst_hbm.at[s, g], dst_v[t])
            pltpu.sync_copy(ew_hbm.at[s, g], w_v[t])

        def fire_gather(t):
            pltpu.async_copy(x2_hbm.at[src_v[t].at[0]], rows[t], gsem[t])

        def drain_gather(t):
            pltpu.make_async_copy(x2_hbm.at[src_v[t].at[0]], rows[t],
                                  gsem[t]).wait()

        def fire_scatter(t):
            pltpu.async_copy(rows[t], accum.at[dst_v[t].at[0]], ssem[t],
                             add=True)

        def drain_scatter(t):
            pltpu.make_async_copy(rows[t], accum.at[dst_v[t].at[0]],
                                  ssem[t]).wait()

        for b in range(2):
            stage(b, b)
            fire_gather(b)

        def pbody(k, _):
            for b in range(_NSLOT):
                j = _NSLOT * k + b
                drain_gather(b)
                scale_chunk(rows[b], b)
                fire_scatter(b)
                nb = (b + 2) % _NSLOT

                @pl.when(j >= 2)
                def _():
                    drain_scatter(nb)

                stage(nb, j + 2)
                fire_gather(nb)
            return 0

        lax.fori_loop(0, (NCHUNK - 2) // _NSLOT, pbody, 0)
        for b in range(2):
            j = NCHUNK - 2 + b
            drain_gather(b)
            scale_chunk(rows[b], b)
            fire_scatter(b)
            drain_scatter((b + 2) % _NSLOT)
        for b in range(2):
            drain_scatter(b)

        plsc.subcore_barrier()
        pltpu.sync_copy(accum.at[pl.ds(s * ROWS_PT, ROWS_PT)],
                        out_hbm.at[pl.ds(s * ROWS_PT, ROWS_PT)])

    return agg


@functools.lru_cache(maxsize=None)
def _make_tc_post(N, D):
    def body(p_ref, w_ref, o_ref):
        y = p_ref[:N]
        z = jnp.dot(y, w_ref[...], preferred_element_type=jnp.float32)
        mean = jnp.mean(z, axis=0, keepdims=True)
        zc = z - mean
        var = jnp.mean(zc * zc, axis=0, keepdims=True)
        zr = jnp.maximum(zc * lax.rsqrt(var + 0.001), 0.0)
        ss = jnp.sum(zr * zr)
        o_ref[...] = zr * lax.rsqrt(jnp.maximum(ss, 1e-12))

    return pl.pallas_call(
        body,
        out_shape=jax.ShapeDtypeStruct((N, D), jnp.float32),
    )


def kernel(x, edge_index, edge_weight, W):
    N, D = x.shape
    E = edge_weight.shape[0]
    EPT = E // _NS
    NCHUNK = EPT // _CH
    src = edge_index[0].reshape(_NS, NCHUNK, 1, _CH)
    dst = edge_index[1].reshape(_NS, NCHUNK, 1, _CH)
    ew = edge_weight.reshape(_NS, NCHUNK, 1, _CH)
    partials = _make_sc_aggregate(N, E, D)(x, src, dst, ew)
    return _make_tc_post(N, D)(partials, W)


# speedup vs baseline: 1.5096x; 1.3008x over previous
import functools

import jax
import jax.numpy as jnp
from jax import lax
from jax.experimental import pallas as pl
from jax.experimental.pallas import tpu as pltpu
from jax.experimental.pallas import tpu_sc as plsc

_NC = 2
_NS = 16
_LANES = 16

_CH = 80
_GRP = 2


@functools.lru_cache(maxsize=None)
def _make_sc_aggregate(N, E, D):
    DH = D
    EPT = E // _NS
    NCHUNK = EPT // _CH
    NGROUP = NCHUNK // _GRP
    assert EPT * _NS == E and NCHUNK * _CH == EPT and NGROUP * _GRP == NCHUNK
    NP = -(-N // (_NS * 128)) * (_NS * 128)
    ROWS_PT = NP // _NS
    assert ROWS_PT % _CH == 0 and DH % _LANES == 0

    mesh = plsc.VectorSubcoreMesh(core_axis_name="c", subcore_axis_name="s",
                                  num_cores=1)

    @functools.partial(
        pl.kernel,
        out_type=jax.ShapeDtypeStruct((NP, DH), jnp.float32),
        mesh=mesh,
        scratch_types=[
            [pltpu.VMEM((_GRP, _CH), jnp.int32) for _ in range(2)],
            [pltpu.VMEM((_GRP, _CH), jnp.int32) for _ in range(2)],
            [pltpu.VMEM((_GRP, _CH), jnp.float32) for _ in range(2)],
            [pltpu.VMEM((_CH, DH), jnp.float32) for _ in range(2 * _GRP)],
            pltpu.VMEM_SHARED((NP, DH), jnp.float32),
            [pltpu.SemaphoreType.DMA for _ in range(2)],
            [pltpu.SemaphoreType.DMA for _ in range(2)],
        ],
    )
    def agg(x2_hbm, src_hbm, dst_hbm, ew_hbm, out_hbm,
            src_v, dst_v, w_v, rows, accum, gsem, ssem):
        c = lax.axis_index("c")
        s = lax.axis_index("s")

        zero = jnp.zeros((_LANES,), jnp.float32)

        def zbody(i, _):
            r = i // (DH // _LANES)
            f = i % (DH // _LANES)
            rows[0][r, pl.ds(f * _LANES, _LANES)] = zero
            return 0

        lax.fori_loop(0, _CH * (DH // _LANES), zbody, 0)
        for j in range(ROWS_PT // _CH):
            pltpu.sync_copy(rows[0],
                            accum.at[pl.ds(s * ROWS_PT + j * _CH, _CH)])
        plsc.subcore_barrier()

        del c

        def scale_chunk(rows_ref, t, wrow):
            def qbody(q, _):
                w16 = w_v[t][wrow, pl.ds(q * _LANES, _LANES)]
                for e in range(_LANES):
                    wv = lax.gather(
                        w16, jnp.full((_LANES, 1), e, jnp.int32),
                        lax.GatherDimensionNumbers(
                            offset_dims=(), collapsed_slice_dims=(0,),
                            start_index_map=(0,)),
                        slice_sizes=(1,),
                        mode=lax.GatherScatterMode.PROMISE_IN_BOUNDS)
                    i = q * _LANES + e
                    for f in range(DH // _LANES):
                        sl = pl.ds(f * _LANES, _LANES)
                        rows_ref[i, sl] = rows_ref[i, sl] * wv
                return 0

            lax.fori_loop(0, _CH // _LANES, qbody, 0)

        def stage(t, g):
            pltpu.sync_copy(src_hbm.at[s, g], src_v[t])
            pltpu.sync_copy(dst_hbm.at[s, g], dst_v[t])
            pltpu.sync_copy(ew_hbm.at[s, g], w_v[t])

        def fire_gathers(t):
            for b in range(_GRP):
                pltpu.async_copy(x2_hbm.at[src_v[t].at[b]],
                                 rows[t * _GRP + b], gsem[t])

        def drain_gathers(t):
            for b in range(_GRP):
                pltpu.make_async_copy(x2_hbm.at[src_v[t].at[b]],
                                      rows[t * _GRP + b], gsem[t]).wait()

        def scale_all(t):
            for b in range(_GRP):
                scale_chunk(rows[t * _GRP + b], t, b)

        def fire_scatters(t):
            for b in range(_GRP):
                pltpu.async_copy(rows[t * _GRP + b],
                                 accum.at[dst_v[t].at[b]], ssem[t],
                                 add=True)

        def drain_scatters(t):
            for b in range(_GRP):
                pltpu.make_async_copy(rows[t * _GRP + b],
                                      accum.at[dst_v[t].at[b]],
                                      ssem[t]).wait()

        NPAIR = (NGROUP - 1) // 2
        stage(0, 0)
        fire_gathers(0)

        def process_set(t):
            for b in range(_GRP):
                pltpu.make_async_copy(x2_hbm.at[src_v[t].at[b]],
                                      rows[t * _GRP + b], gsem[t]).wait()
                scale_chunk(rows[t * _GRP + b], t, b)
                pltpu.async_copy(rows[t * _GRP + b],
                                 accum.at[dst_v[t].at[b]], ssem[t],
                                 add=True)

        def pbody(k, _):
            stage(1, 2 * k + 1)
            fire_gathers(1)
            process_set(0)
            process_set(1)
            drain_scatters(0)
            stage(0, 2 * k + 2)
            fire_gathers(0)
            drain_scatters(1)
            return 0

        lax.fori_loop(0, NPAIR, pbody, 0)
        process_set(0)
        drain_scatters(0)

        plsc.subcore_barrier()
        pltpu.sync_copy(accum.at[pl.ds(s * ROWS_PT, ROWS_PT)],
                        out_hbm.at[pl.ds(s * ROWS_PT, ROWS_PT)])

    return agg


@functools.lru_cache(maxsize=None)
def _make_tc_post(N, D):
    def body(p_ref, w_ref, o_ref):
        y = p_ref[:N]
        z = jnp.dot(y, w_ref[...], preferred_element_type=jnp.float32)
        mean = jnp.mean(z, axis=0, keepdims=True)
        zc = z - mean
        var = jnp.mean(zc * zc, axis=0, keepdims=True)
        zr = jnp.maximum(zc * lax.rsqrt(var + 0.001), 0.0)
        ss = jnp.sum(zr * zr)
        o_ref[...] = zr * lax.rsqrt(jnp.maximum(ss, 1e-12))

    return pl.pallas_call(
        body,
        out_shape=jax.ShapeDtypeStruct((N, D), jnp.float32),
    )


def kernel(x, edge_index, edge_weight, W):
    N, D = x.shape
    E = edge_weight.shape[0]
    EPT = E // _NS
    NCHUNK = EPT // _CH
    NGROUP = NCHUNK // _GRP
    src = edge_index[0].reshape(_NS, NGROUP, _GRP, _CH)
    dst = edge_index[1].reshape(_NS, NGROUP, _GRP, _CH)
    ew = edge_weight.reshape(_NS, NGROUP, _GRP, _CH)
    partials = _make_sc_aggregate(N, E, D)(x, src, dst, ew)
    return _make_tc_post(N, D)(partials, W)
